# Initial kernel scaffold; baseline (speedup 1.0000x reference)
import functools

import jax
import jax.numpy as jnp
from jax import lax
from jax.experimental import pallas as pl
from jax.experimental.pallas import tpu as pltpu
from jax.experimental.pallas import tpu_sc as plsc

EMB = 32
EPS = 1e-12
HALF = 16
NW = 32
CHUNK = 128


def kernel(entity_ids, table, gamma, beta):
    bsz, hist = entity_ids.shape
    nrows = bsz * hist
    rows_per_worker = nrows // NW
    nchunks = rows_per_worker // CHUNK
    ids = entity_ids.astype(jnp.int32).reshape(NW, nchunks, CHUNK)

    mesh = plsc.VectorSubcoreMesh(core_axis_name="c", subcore_axis_name="s")

    @functools.partial(
        pl.kernel,
        out_type=jax.ShapeDtypeStruct((nrows, EMB), jnp.float32),
        mesh=mesh,
        scratch_types=[
            pltpu.VMEM((nchunks, CHUNK), jnp.int32),
            pltpu.VMEM((CHUNK, EMB), jnp.float32),
            pltpu.VMEM((EMB,), jnp.float32),
            pltpu.VMEM((EMB,), jnp.float32),
            pltpu.SemaphoreType.DMA,
        ],
    )
    def sc_kernel(ids_hbm, table_hbm, gamma_hbm, beta_hbm, out_hbm,
                  idx_v, data_v, gam_v, bet_v, sem):
        wid = lax.axis_index("s") * 2 + lax.axis_index("c")
        pltpu.sync_copy(ids_hbm.at[wid], idx_v)
        pltpu.sync_copy(gamma_hbm, gam_v)
        pltpu.sync_copy(beta_hbm, bet_v)
        g0 = gam_v[pl.ds(0, HALF)]
        g1 = gam_v[pl.ds(HALF, HALF)]
        b0 = bet_v[pl.ds(0, HALF)]
        b1 = bet_v[pl.ds(HALF, HALF)]
        base = wid * rows_per_worker

        def chunk_body(c, _):
            pltpu.async_copy(table_hbm.at[idx_v.at[c]], data_v, sem).wait()

            def row_body(r8, _):
                for u in range(8):
                    r = r8 * 8 + u
                    v0 = data_v[r, pl.ds(0, HALF)]
                    v1 = data_v[r, pl.ds(HALF, HALF)]
                    s = jnp.sum(v0 + v1)
                    q = jnp.sum(v0 * v0 + v1 * v1)
                    mean = s * (1.0 / EMB)
                    var = jnp.maximum(q * (1.0 / EMB) - mean * mean, 0.0) + EPS
                    i = lax.bitcast_convert_type(var, jnp.int32)
                    i = jnp.int32(0x5F3759DF) - lax.shift_right_logical(i, 1)
                    y = lax.bitcast_convert_type(i, jnp.float32)
                    xh = var * 0.5
                    y = y * (1.5 - xh * y * y)
                    y = y * (1.5 - xh * y * y)
                    y = y * (1.5 - xh * y * y)
                    data_v[r, pl.ds(0, HALF)] = (v0 - mean) * y * g0 + b0
                    data_v[r, pl.ds(HALF, HALF)] = (v1 - mean) * y * g1 + b1
                return 0

            lax.fori_loop(0, CHUNK // 8, row_body, 0)
            pltpu.sync_copy(data_v, out_hbm.at[pl.ds(base + c * CHUNK, CHUNK)])
            return 0

        lax.fori_loop(0, nchunks, chunk_body, 0)

    out = sc_kernel(ids, table, gamma, beta)
    return out.reshape(bsz, hist, EMB)


# speedup vs baseline: 1.1115x; 1.1115x over previous
import functools

import jax
import jax.numpy as jnp
from jax import lax
from jax.experimental import pallas as pl
from jax.experimental.pallas import tpu as pltpu
from jax.experimental.pallas import tpu_sc as plsc

EMB = 32
EPS = 1e-12
HALF = 16
NW = 32
CHUNK = 128


def kernel(entity_ids, table, gamma, beta):
    bsz, hist = entity_ids.shape
    nrows = bsz * hist
    rows_per_worker = nrows // NW
    nchunks = rows_per_worker // CHUNK
    ids = entity_ids.astype(jnp.int32).reshape(NW, nchunks, CHUNK)

    mesh = plsc.VectorSubcoreMesh(core_axis_name="c", subcore_axis_name="s")

    @functools.partial(
        pl.kernel,
        out_type=jax.ShapeDtypeStruct((nrows, EMB), jnp.float32),
        mesh=mesh,
        scratch_types=[
            pltpu.VMEM((nchunks, CHUNK), jnp.int32),
            pltpu.VMEM((CHUNK, EMB), jnp.float32),
            pltpu.VMEM((EMB,), jnp.float32),
            pltpu.VMEM((EMB,), jnp.float32),
            pltpu.SemaphoreType.DMA,
        ],
        compiler_params=pltpu.CompilerParams(
            needs_layout_passes=False, use_tc_tiling_on_sc=False),
    )
    def sc_kernel(ids_hbm, table_hbm, gamma_hbm, beta_hbm, out_hbm,
                  idx_v, data_v, gam_v, bet_v, sem):
        wid = lax.axis_index("s") * 2 + lax.axis_index("c")
        pltpu.sync_copy(ids_hbm.at[wid], idx_v)
        pltpu.sync_copy(gamma_hbm, gam_v)
        pltpu.sync_copy(beta_hbm, bet_v)
        g0 = gam_v[pl.ds(0, HALF)]
        g1 = gam_v[pl.ds(HALF, HALF)]
        b0 = bet_v[pl.ds(0, HALF)]
        b1 = bet_v[pl.ds(HALF, HALF)]
        base = wid * rows_per_worker

        def chunk_body(c, _):
            pltpu.async_copy(table_hbm.at[idx_v.at[c]], data_v, sem).wait()

            def row_body(r8, _):
                for u in range(8):
                    r = r8 * 8 + u
                    v0 = data_v[r, pl.ds(0, HALF)]
                    v1 = data_v[r, pl.ds(HALF, HALF)]
                    s = jnp.sum(v0 + v1)
                    q = jnp.sum(v0 * v0 + v1 * v1)
                    mean = s * (1.0 / EMB)
                    var = jnp.maximum(q * (1.0 / EMB) - mean * mean, 0.0) + EPS
                    i = lax.bitcast_convert_type(var, jnp.int32)
                    i = jnp.int32(0x5F3759DF) - lax.shift_right_logical(i, 1)
                    y = lax.bitcast_convert_type(i, jnp.float32)
                    xh = var * 0.5
                    y = y * (1.5 - xh * y * y)
                    y = y * (1.5 - xh * y * y)
                    y = y * (1.5 - xh * y * y)
                    data_v[r, pl.ds(0, HALF)] = (v0 - mean) * y * g0 + b0
                    data_v[r, pl.ds(HALF, HALF)] = (v1 - mean) * y * g1 + b1
                return 0

            lax.fori_loop(0, CHUNK // 8, row_body, 0)
            pltpu.sync_copy(data_v, out_hbm.at[pl.ds(base + c * CHUNK, CHUNK)])
            return 0

        lax.fori_loop(0, nchunks, chunk_body, 0)

    out = sc_kernel(ids, table, gamma, beta)
    return out.reshape(bsz, hist, EMB)


# speedup vs baseline: 1.3623x; 1.2256x over previous
import functools

import jax
import jax.numpy as jnp
from jax import lax
from jax.experimental import pallas as pl
from jax.experimental.pallas import tpu as pltpu
from jax.experimental.pallas import tpu_sc as plsc

EMB = 32
EPS = 1e-12
HALF = 16
NW = 32
CHUNK = 128
NBUF = 4


def kernel(entity_ids, table, gamma, beta):
    bsz, hist = entity_ids.shape
    nrows = bsz * hist
    rows_per_worker = nrows // NW
    nchunks = rows_per_worker // CHUNK
    ngroups = nchunks // NBUF
    ids = entity_ids.astype(jnp.int32).reshape(NW, nchunks, CHUNK)

    mesh = plsc.VectorSubcoreMesh(core_axis_name="c", subcore_axis_name="s")

    @functools.partial(
        pl.kernel,
        out_type=jax.ShapeDtypeStruct((nrows, EMB), jnp.float32),
        mesh=mesh,
        scratch_types=[
            pltpu.VMEM((nchunks, CHUNK), jnp.int32),
            pltpu.VMEM((NBUF, CHUNK, EMB), jnp.float32),
            pltpu.VMEM((EMB,), jnp.float32),
            pltpu.VMEM((EMB,), jnp.float32),
        ] + [pltpu.SemaphoreType.DMA] * (2 * NBUF),
        compiler_params=pltpu.CompilerParams(
            needs_layout_passes=False, use_tc_tiling_on_sc=False),
    )
    def sc_kernel(ids_hbm, table_hbm, gamma_hbm, beta_hbm, out_hbm,
                  idx_v, data_v, gam_v, bet_v, *sems):
        gsem = sems[:NBUF]
        ssem = sems[NBUF:]
        wid = lax.axis_index("s") * 2 + lax.axis_index("c")
        pltpu.sync_copy(ids_hbm.at[wid], idx_v)
        pltpu.sync_copy(gamma_hbm, gam_v)
        pltpu.sync_copy(beta_hbm, bet_v)
        g0 = gam_v[pl.ds(0, HALF)]
        g1 = gam_v[pl.ds(HALF, HALF)]
        b0 = bet_v[pl.ds(0, HALF)]
        b1 = bet_v[pl.ds(HALF, HALF)]
        base = wid * rows_per_worker
        lane15 = jnp.full((HALF, 1), 15, jnp.int32)
        dnums = lax.GatherDimensionNumbers(
            offset_dims=(), collapsed_slice_dims=(0,), start_index_map=(0,))

        def bcast_last(x):
            return lax.gather(x, lane15, dnums, (1,),
                              mode=lax.GatherScatterMode.PROMISE_IN_BOUNDS)

        def ln_chunk(b):
            def row(r):
                v0 = data_v[b, r, pl.ds(0, HALF)]
                v1 = data_v[b, r, pl.ds(HALF, HALF)]
                s = bcast_last(jnp.cumsum(v0 + v1))
                q = bcast_last(jnp.cumsum(v0 * v0 + v1 * v1))
                mean = s * (1.0 / EMB)
                var = jnp.maximum(q * (1.0 / EMB) - mean * mean, 0.0) + EPS
                i = lax.bitcast_convert_type(var, jnp.int32)
                i = jnp.int32(0x5F3759DF) - lax.shift_right_logical(i, 1)
                y = lax.bitcast_convert_type(i, jnp.float32)
                xh = var * 0.5
                y = y * (1.5 - xh * y * y)
                y = y * (1.5 - xh * y * y)
                y = y * (1.5 - xh * y * y)
                data_v[b, r, pl.ds(0, HALF)] = (v0 - mean) * (y * g0) + b0
                data_v[b, r, pl.ds(HALF, HALF)] = (v1 - mean) * (y * g1) + b1

            plsc.parallel_loop(0, CHUNK, 1, unroll=8)(row)

        def start_gather(c, b):
            return pltpu.async_copy(table_hbm.at[idx_v.at[c]],
                                    data_v.at[b], gsem[b])

        def start_store(c, b):
            return pltpu.async_copy(
                data_v.at[b], out_hbm.at[pl.ds(base + c * CHUNK, CHUNK)],
                ssem[b])

        for b in range(NBUF):
            start_gather(b, b)

        def group(p, _):
            for b in range(NBUF):
                c = p * NBUF + b
                pltpu.make_async_copy(table_hbm.at[idx_v.at[c]],
                                      data_v.at[b], gsem[b]).wait()
                ln_chunk(b)
                start_store(c, b)
            for b in range(NBUF):
                c = p * NBUF + b
                pltpu.make_async_copy(
                    data_v.at[b],
                    out_hbm.at[pl.ds(base + c * CHUNK, CHUNK)],
                    ssem[b]).wait()
                start_gather(c + NBUF, b)
            return 0

        lax.fori_loop(0, ngroups - 1, group, 0)

        for b in range(NBUF):
            c = nchunks - NBUF + b
            pltpu.make_async_copy(table_hbm.at[idx_v.at[c]],
                                  data_v.at[b], gsem[b]).wait()
            ln_chunk(b)
            start_store(c, b)
        for b in range(NBUF):
            c = nchunks - NBUF + b
            pltpu.make_async_copy(
                data_v.at[b],
                out_hbm.at[pl.ds(base + c * CHUNK, CHUNK)],
                ssem[b]).wait()

    out = sc_kernel(ids, table, gamma, beta)
    return out.reshape(bsz, hist, EMB)


# speedup vs baseline: 1.4304x; 1.0500x over previous
import functools

import jax
import jax.numpy as jnp
from jax import lax
from jax.experimental import pallas as pl
from jax.experimental.pallas import tpu as pltpu
from jax.experimental.pallas import tpu_sc as plsc

EMB = 32
EPS = 1e-12
HALF = 16
NW = 32
CHUNK = 128
UNIT_B = 1024


def kernel(entity_ids, table, gamma, beta):
    bsz, hist = entity_ids.shape
    nrows = bsz * hist
    nunits = hist * (bsz // UNIT_B)
    units_per_worker = nunits // NW
    chunks_per_unit = UNIT_B // CHUNK
    bhi_per_b = bsz // 128
    h_stride = EMB * bsz
    ids_t = entity_ids.astype(jnp.int32).T

    mesh = plsc.VectorSubcoreMesh(core_axis_name="c", subcore_axis_name="s")

    @functools.partial(
        pl.kernel,
        out_type=jax.ShapeDtypeStruct((nrows * EMB,), jnp.float32),
        mesh=mesh,
        scratch_types=[
            pltpu.VMEM((UNIT_B,), jnp.int32),
            pltpu.VMEM((2, CHUNK, EMB), jnp.float32),
            pltpu.VMEM((4 * 8 * UNIT_B,), jnp.float32),
            pltpu.VMEM((EMB,), jnp.float32),
            pltpu.VMEM((EMB,), jnp.float32),
        ] + [pltpu.SemaphoreType.DMA] * 3,
        compiler_params=pltpu.CompilerParams(
            needs_layout_passes=False, use_tc_tiling_on_sc=False),
    )
    def sc_kernel(ids_hbm, table_hbm, gamma_hbm, beta_hbm, out_hbm,
                  idx_v, data_v, stage_v, gam_v, bet_v,
                  gsem0, gsem1, ssem):
        gsem = (gsem0, gsem1)
        wid = lax.axis_index("s") * 2 + lax.axis_index("c")
        pltpu.sync_copy(gamma_hbm, gam_v)
        pltpu.sync_copy(beta_hbm, bet_v)
        g0 = gam_v[pl.ds(0, HALF)]
        g1 = gam_v[pl.ds(HALF, HALF)]
        b0 = bet_v[pl.ds(0, HALF)]
        b1 = bet_v[pl.ds(HALF, HALF)]
        lane15 = jnp.full((HALF, 1), 15, jnp.int32)
        dnums = lax.GatherDimensionNumbers(
            offset_dims=(), collapsed_slice_dims=(0,), start_index_map=(0,))

        def bcast_last(x):
            return lax.gather(x, lane15, dnums, (1,),
                              mode=lax.GatherScatterMode.PROMISE_IN_BOUNDS)

        iota = lax.iota(jnp.int32, HALF)
        k01 = (iota // 8) * (8 * UNIT_B) + (iota % 8) * 128
        k23 = k01 + 2 * (8 * UNIT_B)

        def start_gather(k, slot):
            return pltpu.async_copy(
                table_hbm.at[idx_v.at[pl.ds(k * CHUNK, CHUNK)]],
                data_v.at[slot], gsem[slot])

        def unit_body(u, _):
            h = u // (bhi_per_b // 8)
            q = u % (bhi_per_b // 8)
            pltpu.sync_copy(ids_hbm.at[h, pl.ds(q * UNIT_B, UNIT_B)], idx_v)
            start_gather(0, 0)
            for k in range(chunks_per_unit):
                slot = k % 2
                if k + 1 < chunks_per_unit:
                    start_gather(k + 1, (k + 1) % 2)
                pltpu.make_async_copy(
                    table_hbm.at[idx_v.at[pl.ds(k * CHUNK, CHUNK)]],
                    data_v.at[slot], gsem[slot]).wait()

                def row(r):
                    v0 = data_v[slot, r, pl.ds(0, HALF)]
                    v1 = data_v[slot, r, pl.ds(HALF, HALF)]
                    s = bcast_last(jnp.cumsum(v0 + v1))
                    q2 = bcast_last(jnp.cumsum(v0 * v0 + v1 * v1))
                    mean = s * (1.0 / EMB)
                    var = jnp.maximum(
                        q2 * (1.0 / EMB) - mean * mean, 0.0) + EPS
                    i = lax.bitcast_convert_type(var, jnp.int32)
                    i = (jnp.int32(0x5F3759DF)
                         - lax.shift_right_logical(i, 1))
                    y = lax.bitcast_convert_type(i, jnp.float32)
                    xh = var * 0.5
                    y = y * (1.5 - xh * y * y)
                    y = y * (1.5 - xh * y * y)
                    y = y * (1.5 - xh * y * y)
                    pos = k * 1024 + r
                    plsc.store_scatter(stage_v, [k01 + pos],
                                       (v0 - mean) * (y * g0) + b0)
                    plsc.store_scatter(stage_v, [k23 + pos],
                                       (v1 - mean) * (y * g1) + b1)

                plsc.parallel_loop(0, CHUNK, 1, unroll=8)(row)

            out_off = h * h_stride + q * (8 * UNIT_B)
            for ch in range(4):
                pltpu.async_copy(
                    stage_v.at[pl.ds(ch * 8 * UNIT_B, 8 * UNIT_B)],
                    out_hbm.at[pl.ds(out_off + ch * (8 * bsz), 8 * UNIT_B)],
                    ssem)
            for ch in range(4):
                pltpu.make_async_copy(
                    stage_v.at[pl.ds(ch * 8 * UNIT_B, 8 * UNIT_B)],
                    out_hbm.at[pl.ds(out_off + ch * (8 * bsz), 8 * UNIT_B)],
                    ssem).wait()
            return 0

        lax.fori_loop(wid * units_per_worker, (wid + 1) * units_per_worker,
                      unit_body, 0)

    out_flat = sc_kernel(ids_t, table, gamma, beta)
    out5 = out_flat.reshape(hist, 4, bhi_per_b, 8, 128)
    return out5.transpose(2, 4, 0, 1, 3).reshape(bsz, hist, EMB)


# speedup vs baseline: 1.4378x; 1.0052x over previous
import functools

import jax
import jax.numpy as jnp
from jax import lax
from jax.experimental import pallas as pl
from jax.experimental.pallas import tpu as pltpu
from jax.experimental.pallas import tpu_sc as plsc

EMB = 32
EPS = 1e-12
HALF = 16
NW = 32
CHUNK = 128
PITCH = 33
UNIT_B = 1024
STG = 4 * 8 * UNIT_B


def kernel(entity_ids, table, gamma, beta):
    bsz, hist = entity_ids.shape
    nrows = bsz * hist
    nunits = hist * (bsz // UNIT_B)
    units_per_worker = nunits // NW
    chunks_per_unit = UNIT_B // CHUNK
    h_stride = EMB * bsz
    ids_t = entity_ids.astype(jnp.int32).T

    mesh = plsc.VectorSubcoreMesh(core_axis_name="c", subcore_axis_name="s")

    @functools.partial(
        pl.kernel,
        out_type=jax.ShapeDtypeStruct((nrows * EMB,), jnp.float32),
        mesh=mesh,
        scratch_types=[
            pltpu.VMEM((UNIT_B,), jnp.int32),
            pltpu.VMEM((2, CHUNK, EMB), jnp.float32),
            pltpu.VMEM((CHUNK * PITCH,), jnp.float32),
            pltpu.VMEM((2 * STG,), jnp.float32),
            pltpu.VMEM((EMB,), jnp.float32),
            pltpu.VMEM((EMB,), jnp.float32),
            pltpu.VMEM((EMB * HALF,), jnp.float32),
            pltpu.VMEM((EMB * HALF,), jnp.float32),
            pltpu.SemaphoreType.DMA,
            pltpu.SemaphoreType.DMA,
            pltpu.SemaphoreType.DMA((2,)),
        ],
        compiler_params=pltpu.CompilerParams(
            needs_layout_passes=False, use_tc_tiling_on_sc=False),
    )
    def sc_kernel(ids_hbm, table_hbm, gamma_hbm, beta_hbm, out_hbm,
                  idx_v, data_v, pad_v, stage_v, gam_v, bet_v, gsp_v, bsp_v,
                  gsem0, gsem1, ssem):
        gsem = (gsem0, gsem1)
        wid = lax.axis_index("s") * 2 + lax.axis_index("c")
        pltpu.sync_copy(gamma_hbm, gam_v)
        pltpu.sync_copy(beta_hbm, bet_v)
        for half in range(2):
            gh = gam_v[pl.ds(half * HALF, HALF)]
            bh = bet_v[pl.ds(half * HALF, HALF)]
            for j in range(HALF):
                c = half * HALF + j
                gsp_v[pl.ds(c * HALF, HALF)] = jnp.full(
                    (HALF,), gh[j], jnp.float32)
                bsp_v[pl.ds(c * HALF, HALF)] = jnp.full(
                    (HALF,), bh[j], jnp.float32)
        iota = lax.iota(jnp.int32, HALF)
        iota_p = iota * PITCH
        lo = wid * units_per_worker

        def start_gather(k, slot):
            return pltpu.async_copy(
                table_hbm.at[idx_v.at[pl.ds(k * CHUNK, CHUNK)]],
                data_v.at[slot], gsem[slot])

        def stores(su, out_off, wait):
            for ch in range(4):
                cp = pltpu.make_async_copy(
                    stage_v.at[pl.ds(su * STG + ch * 8 * UNIT_B, 8 * UNIT_B)],
                    out_hbm.at[pl.ds(out_off + ch * (8 * bsz), 8 * UNIT_B)],
                    ssem.at[su])
                if wait:
                    cp.wait()
                else:
                    cp.start()

        def unit_body(u, _):
            h = u // (bsz // UNIT_B)
            q = u % (bsz // UNIT_B)
            su = (u - lo) % 2
            pltpu.sync_copy(ids_hbm.at[h, pl.ds(q * UNIT_B, UNIT_B)], idx_v)

            @pl.when(u - lo >= 2)
            def _():
                stores(su, 0, wait=True)

            start_gather(0, 0)
            for k in range(chunks_per_unit):
                slot = k % 2
                if k + 1 < chunks_per_unit:
                    start_gather(k + 1, (k + 1) % 2)
                pltpu.make_async_copy(
                    table_hbm.at[idx_v.at[pl.ds(k * CHUNK, CHUNK)]],
                    data_v.at[slot], gsem[slot]).wait()

                def repack(r):
                    pad_v[pl.ds(r * PITCH, HALF)] = \
                        data_v[slot, r, pl.ds(0, HALF)]
                    pad_v[pl.ds(r * PITCH + HALF, HALF)] = \
                        data_v[slot, r, pl.ds(HALF, HALF)]

                plsc.parallel_loop(0, CHUNK, 1, unroll=8)(repack)

                def group(g, _):
                    base = g * (HALF * PITCH)
                    col = iota_p + base
                    s = jnp.zeros((HALF,), jnp.float32)
                    q2 = jnp.zeros((HALF,), jnp.float32)
                    for c in range(EMB):
                        v = plsc.load_gather(pad_v, [col + c])
                        s = s + v
                        q2 = q2 + v * v
                    mean = s * (1.0 / EMB)
                    var = jnp.maximum(
                        q2 * (1.0 / EMB) - mean * mean, 0.0) + EPS
                    i = lax.bitcast_convert_type(var, jnp.int32)
                    i = (jnp.int32(0x5F3759DF)
                         - lax.shift_right_logical(i, 1))
                    y = lax.bitcast_convert_type(i, jnp.float32)
                    xh = var * 0.5
                    y = y * (1.5 - xh * y * y)
                    y = y * (1.5 - xh * y * y)
                    y = y * (1.5 - xh * y * y)
                    pos = su * STG + k * 1024 + g * HALF
                    for c in range(EMB):
                        v = plsc.load_gather(pad_v, [col + c])
                        gsv = gsp_v[pl.ds(c * HALF, HALF)]
                        bsv = bsp_v[pl.ds(c * HALF, HALF)]
                        o = (v - mean) * (y * gsv) + bsv
                        stage_v[pl.ds(
                            pos + (c // 8) * (8 * UNIT_B) + (c % 8) * 128,
                            HALF)] = o
                    return 0

                lax.fori_loop(0, CHUNK // HALF, group, 0)

            out_off = h * h_stride + q * (8 * UNIT_B)
            stores(su, out_off, wait=False)
            return 0

        lax.fori_loop(lo, lo + units_per_worker, unit_body, 0)

        for su in range(2):
            stores(su, 0, wait=True)

    out_flat = sc_kernel(ids_t, table, gamma, beta)
    out5 = out_flat.reshape(hist, 4, bsz // 128, 8, 128)
    return out5.transpose(2, 4, 0, 1, 3).reshape(bsz, hist, EMB)
